# baseline (speedup 1.0000x reference)
import jax
import jax.numpy as jnp
from jax.experimental import pallas as pl
from jax.experimental.pallas import tpu as pltpu

_SIGMA = 0.01
_TOTAL_COUNT = 10

_NOISE = _SIGMA * 0.0

_ROWS = 100000
_COLS = 128
_CR = 4000
_NCHUNK = _ROWS // _CR
_NBUF = 8


def _stream_add_kernel(x_hbm, o_hbm, buf_ref, in_sems, out_sems):
    def cin(t, s):
        return pltpu.make_async_copy(
            x_hbm.at[pl.ds(t * _CR, _CR), :], buf_ref.at[s], in_sems.at[s])

    def cout(t, s):
        return pltpu.make_async_copy(
            buf_ref.at[s], o_hbm.at[pl.ds(t * _CR, _CR), :], out_sems.at[s])

    for i in range(_NBUF):
        cin(i, i).start()
    for t in range(_NCHUNK):
        s = t % _NBUF
        cin(t, s).wait()
        buf_ref[s] = buf_ref[s] + _NOISE
        cout(t, s).start()
        v = t + _NBUF // 2
        if _NBUF <= v < _NCHUNK:
            sv = v % _NBUF
            cout(v - _NBUF, sv).wait()
            cin(v, sv).start()
    for t in range(_NCHUNK - _NBUF, _NCHUNK):
        cout(t, t % _NBUF).wait()


def kernel(X):
    out_t = pl.pallas_call(
        _stream_add_kernel,
        in_specs=[pl.BlockSpec(memory_space=pltpu.HBM)],
        out_specs=pl.BlockSpec(memory_space=pltpu.HBM),
        out_shape=jax.ShapeDtypeStruct((_ROWS, _COLS), X.dtype),
        scratch_shapes=[
            pltpu.VMEM((_NBUF, _CR, _COLS), jnp.float32),
            pltpu.SemaphoreType.DMA((_NBUF,)),
            pltpu.SemaphoreType.DMA((_NBUF,)),
        ],
    )(X.T)
    return out_t.T


# speedup vs baseline: 1.0042x; 1.0042x over previous
import jax
import jax.numpy as jnp
from jax.experimental import pallas as pl
from jax.experimental.pallas import tpu as pltpu

_SIGMA = 0.01
_TOTAL_COUNT = 10

_NOISE = _SIGMA * 0.0

_ROWS = 100000
_COLS = 128
_CR = 10000
_NCHUNK = _ROWS // _CR
_NBUF = 6


def _stream_add_kernel(x_hbm, o_hbm, buf_ref, in_sems, out_sems):
    def cin(t, s):
        return pltpu.make_async_copy(
            x_hbm.at[pl.ds(t * _CR, _CR), :], buf_ref.at[s], in_sems.at[s])

    def cout(t, s):
        return pltpu.make_async_copy(
            buf_ref.at[s], o_hbm.at[pl.ds(t * _CR, _CR), :], out_sems.at[s])

    for i in range(_NBUF):
        cin(i, i).start()
    for t in range(_NCHUNK):
        s = t % _NBUF
        cin(t, s).wait()
        buf_ref[s] = buf_ref[s] + _NOISE
        cout(t, s).start()
        v = t + _NBUF // 2
        if _NBUF <= v < _NCHUNK:
            sv = v % _NBUF
            cout(v - _NBUF, sv).wait()
            cin(v, sv).start()
    for t in range(_NCHUNK - _NBUF, _NCHUNK):
        cout(t, t % _NBUF).wait()


def kernel(X):
    out_t = pl.pallas_call(
        _stream_add_kernel,
        in_specs=[pl.BlockSpec(memory_space=pltpu.HBM)],
        out_specs=pl.BlockSpec(memory_space=pltpu.HBM),
        out_shape=jax.ShapeDtypeStruct((_ROWS, _COLS), X.dtype),
        scratch_shapes=[
            pltpu.VMEM((_NBUF, _CR, _COLS), jnp.float32),
            pltpu.SemaphoreType.DMA((_NBUF,)),
            pltpu.SemaphoreType.DMA((_NBUF,)),
        ],
    )(X.T)
    return out_t.T
